# baseline (speedup 1.0000x reference)
import functools

import jax
import jax.numpy as jnp
from jax import lax
from jax.experimental import pallas as pl
from jax.experimental.pallas import tpu as pltpu
from jax.experimental.pallas import tpu_sc as plsc

B, H, W = 4, 384, 384
CIN, CF, NCLS = 3, 96, 10
MAXLEN = 512
NTOK = 512
ROWS = 32
P = ROWS * W
NT = H // ROWS

NPIX = B * H * W
WORKERS = 32
PPW = NPIX // WORKERS
CHUNK = 2048
NCHUNK = PPW // CHUNK
TBL = NCLS * NTOK


def _sc_body(seg_hbm, gts_hbm, sgout_hbm, lbl_hbm,
             seg_v, gts_v, sgo_v, table_v, red_v, lbl_v, shared):
    c = lax.axis_index("c")
    s = lax.axis_index("s")
    img_id = 2 * c + s // 8
    pix_base = img_id * (H * W) + (s % 8) * PPW

    def _zero(i, _):
        table_v[pl.ds(i * 16, 16)] = jnp.zeros((16,), jnp.float32)
        return 0
    lax.fori_loop(0, TBL // 16, _zero, 0)

    ones16 = jnp.ones((16,), jnp.float32)
    off1 = img_id * NTOK + 1

    def _chunk(k, _):
        base = pix_base + k * CHUNK
        pltpu.sync_copy(seg_hbm.at[pl.ds(base, CHUNK)], seg_v)
        pltpu.sync_copy(gts_hbm.at[pl.ds(base, CHUNK)], gts_v)

        def _pix(i, _):
            sv = seg_v[pl.ds(i * 16, 16)]
            gv = gts_v[pl.ds(i * 16, 16)]
            sgo_v[pl.ds(i * 16, 16)] = sv + off1
            plsc.addupdate_scatter(table_v, [gv * NTOK + sv], ones16)
            return 0
        lax.fori_loop(0, CHUNK // 16, _pix, 0)
        pltpu.sync_copy(sgo_v, sgout_hbm.at[pl.ds(base, CHUNK)])
        return 0
    lax.fori_loop(0, NCHUNK, _chunk, 0)

    pltpu.sync_copy(table_v, shared.at[s])
    plsc.subcore_barrier()

    @pl.when((s == 0) | (s == 8))
    def _reduce():
        def _slot(t, _):
            pltpu.sync_copy(shared.at[s + 1 + t], red_v)

            def _acc(i, _):
                d = pl.ds(i * 16, 16)
                table_v[d] += red_v[d]
                return 0
            lax.fori_loop(0, TBL // 16, _acc, 0)
            return 0
        lax.fori_loop(0, 7, _slot, 0)

        def _lbl(j, _):
            d = pl.ds(j * 16, 16)
            best = table_v[d]
            besti = jnp.zeros((16,), jnp.int32)
            for cc in range(1, NCLS):
                v = table_v[pl.ds(cc * NTOK + j * 16, 16)]
                besti = jnp.where(v > best, cc, besti)
                best = jnp.maximum(v, best)
            lbl_v[d] = besti.astype(jnp.float32)
            return 0
        lax.fori_loop(0, NTOK // 16, _lbl, 0)
        pltpu.sync_copy(lbl_v, lbl_hbm.at[pl.ds(img_id * NTOK, NTOK)])


def _sc_call(seg_flat, gts_flat):
    mesh = plsc.VectorSubcoreMesh(core_axis_name="c", subcore_axis_name="s")
    fn = pl.kernel(
        _sc_body, mesh=mesh,
        compiler_params=pltpu.CompilerParams(needs_layout_passes=False),
        out_type=[
            jax.ShapeDtypeStruct((NPIX,), jnp.int32),
            jax.ShapeDtypeStruct((B * NTOK,), jnp.float32),
        ],
        scratch_types=[
            pltpu.VMEM((CHUNK,), jnp.int32),
            pltpu.VMEM((CHUNK,), jnp.int32),
            pltpu.VMEM((CHUNK,), jnp.int32),
            pltpu.VMEM((TBL,), jnp.float32),
            pltpu.VMEM((TBL,), jnp.float32),
            pltpu.VMEM((NTOK,), jnp.float32),
            pltpu.VMEM_SHARED((16, TBL), jnp.float32),
        ],
    )
    return fn(seg_flat, gts_flat)


def _main_body(img_ref, seg_ref, w1_ref, w2_ref,
               cnn_ref, tok_ref, acc_ref):
    t = pl.program_id(1)

    @pl.when(t == 0)
    def _init():
        acc_ref[...] = jnp.zeros_like(acc_ref)

    img_r = img_ref[0].reshape(CIN, P).astype(jnp.bfloat16)
    feats_t = jax.nn.relu(
        jax.lax.dot_general(w1_ref[...].astype(jnp.bfloat16), img_r,
                            (((0,), (0,)), ((), ())),
                            preferred_element_type=jnp.float32))
    cnn = jax.lax.dot_general(w2_ref[...], feats_t,
                              (((0,), (0,)), ((), ())),
                              preferred_element_type=jnp.float32)
    cnn_ref[0] = cnn.reshape(NCLS, ROWS, W)

    seg_row = seg_ref[0].reshape(1, P)
    iota_s = jax.lax.broadcasted_iota(jnp.int32, (NTOK, 1), 0)
    onehot = (iota_s == seg_row).astype(jnp.float8_e4m3fn)

    feats_b = feats_t.astype(jnp.float8_e4m3fn)
    ones_row = jnp.ones((1, P), jnp.float8_e4m3fn)
    rhs = jnp.concatenate([feats_b, ones_row], axis=0)
    acc_ref[...] += jax.lax.dot_general(
        onehot, rhs, (((1,), (1,)), ((), ())),
        preferred_element_type=jnp.float32)

    @pl.when(t == NT - 1)
    def _finalize():
        counts = acc_ref[:, CF:CF + 1]
        tok_ref[0] = acc_ref[:, :CF] / jnp.maximum(counts, 1.0)


def _attn_body(tok_ref, valid_ref, wq_ref, wk_ref, wv_ref, wo_ref, out_ref):
    tok = tok_ref[0]
    q = jnp.dot(tok, wq_ref[...], preferred_element_type=jnp.float32)
    k = jnp.dot(tok, wk_ref[...], preferred_element_type=jnp.float32)
    v = jnp.dot(tok, wv_ref[...], preferred_element_type=jnp.float32)
    s = jax.lax.dot_general(q, k, (((1,), (1,)), ((), ())),
                            preferred_element_type=jnp.float32)
    s = s * (1.0 / jnp.sqrt(jnp.float32(CF)))
    m = jnp.max(s, axis=1, keepdims=True)
    e = jnp.exp(s - m)
    p = e / jnp.sum(e, axis=1, keepdims=True)
    valid = valid_ref[0, 0]
    p = p * valid.reshape(MAXLEN, 1) * valid.reshape(1, MAXLEN)
    ctx = jnp.dot(p, v, preferred_element_type=jnp.float32)
    out_ref[0] = jnp.dot(ctx, wo_ref[...],
                         preferred_element_type=jnp.float32)


def kernel(img, gts, segments, n_tokens, W1, W2, Wq, Wk, Wv, Wo):
    sg_flat, lbl_flat = _sc_call(segments.reshape(-1), gts.reshape(-1))
    seg_global = sg_flat.reshape(B, H, W)
    super_labels = lbl_flat.reshape(B, NTOK)

    cnn_logits, tokens = pl.pallas_call(
        _main_body,
        grid=(B, NT),
        in_specs=[
            pl.BlockSpec((1, CIN, ROWS, W), lambda b, t: (b, 0, t, 0)),
            pl.BlockSpec((1, ROWS, W), lambda b, t: (b, t, 0)),
            pl.BlockSpec((CIN, CF), lambda b, t: (0, 0)),
            pl.BlockSpec((CF, NCLS), lambda b, t: (0, 0)),
        ],
        out_specs=[
            pl.BlockSpec((1, NCLS, ROWS, W), lambda b, t: (b, 0, t, 0)),
            pl.BlockSpec((1, NTOK, CF), lambda b, t: (b, 0, 0)),
        ],
        out_shape=[
            jax.ShapeDtypeStruct((B, NCLS, H, W), jnp.float32),
            jax.ShapeDtypeStruct((B, NTOK, CF), jnp.float32),
        ],
        scratch_shapes=[
            pltpu.VMEM((NTOK, CF + 1), jnp.float32),
        ],
    )(img, segments, W1, W2)

    valid = (jnp.arange(MAXLEN)[None, :] < n_tokens[:, None]).astype(jnp.float32)

    trans_logits = pl.pallas_call(
        _attn_body,
        grid=(B,),
        in_specs=[
            pl.BlockSpec((1, MAXLEN, CF), lambda b: (b, 0, 0)),
            pl.BlockSpec((1, 1, MAXLEN), lambda b: (b, 0, 0)),
            pl.BlockSpec((CF, CF), lambda b: (0, 0)),
            pl.BlockSpec((CF, CF), lambda b: (0, 0)),
            pl.BlockSpec((CF, CF), lambda b: (0, 0)),
            pl.BlockSpec((CF, NCLS), lambda b: (0, 0)),
        ],
        out_specs=pl.BlockSpec((1, MAXLEN, NCLS), lambda b: (b, 0, 0)),
        out_shape=jax.ShapeDtypeStruct((B, MAXLEN, NCLS), jnp.float32),
    )(tokens, valid.reshape(B, 1, MAXLEN), Wq, Wk, Wv, Wo)

    tokens_ids = jnp.arange(1, B * NTOK + 1)
    return (cnn_logits, trans_logits, super_labels, valid, tokens_ids,
            seg_global)


# speedup vs baseline: 1.0007x; 1.0007x over previous
import functools

import jax
import jax.numpy as jnp
from jax import lax
from jax.experimental import pallas as pl
from jax.experimental.pallas import tpu as pltpu
from jax.experimental.pallas import tpu_sc as plsc

B, H, W = 4, 384, 384
CIN, CF, NCLS = 3, 96, 10
MAXLEN = 512
NTOK = 512
ROWS = 32
P = ROWS * W
NT = H // ROWS

NPIX = B * H * W
WORKERS = 32
PPW = NPIX // WORKERS
CHUNK = 9216
NCHUNK = PPW // CHUNK
CPAD = 16
TBL = NTOK * CPAD
SEGW = NTOK // 8


def _sc_body(seg_hbm, gts_hbm, sgout_hbm, lbl_hbm,
             seg_v, gts_v, sgo_v, table_v, acc_v, slot_v, lbl_v, shared):
    c = lax.axis_index("c")
    s = lax.axis_index("s")
    img_id = 2 * c + s // 8
    pix_base = img_id * (H * W) + (s % 8) * PPW

    def _zero(i, _):
        table_v[pl.ds(i * 16, 16)] = jnp.zeros((16,), jnp.float32)
        return 0
    lax.fori_loop(0, TBL // 16, _zero, 0, unroll=8)

    ones16 = jnp.ones((16,), jnp.float32)
    off1 = img_id * NTOK + 1

    def _chunk(k, _):
        base = pix_base + k * CHUNK
        pltpu.sync_copy(seg_hbm.at[pl.ds(base, CHUNK)], seg_v)
        pltpu.sync_copy(gts_hbm.at[pl.ds(base, CHUNK)], gts_v)

        def _pix(i, _):
            sv = seg_v[pl.ds(i * 16, 16)]
            gv = gts_v[pl.ds(i * 16, 16)]
            sgo_v[pl.ds(i * 16, 16)] = sv + off1
            plsc.addupdate_scatter(table_v, [sv * CPAD + gv], ones16)
            return 0
        lax.fori_loop(0, CHUNK // 16, _pix, 0, unroll=8)
        pltpu.sync_copy(sgo_v, sgout_hbm.at[pl.ds(base, CHUNK)])
        return 0
    lax.fori_loop(0, NCHUNK, _chunk, 0)

    pltpu.sync_copy(table_v, shared.at[s])
    plsc.subcore_barrier()

    srcbase = (s // 8) * 8
    segoff = (s % 8) * SEGW * CPAD
    pltpu.sync_copy(shared.at[srcbase, pl.ds(segoff, SEGW * CPAD)], acc_v)

    def _slot(t, _):
        pltpu.sync_copy(shared.at[srcbase + 1 + t, pl.ds(segoff, SEGW * CPAD)],
                        slot_v)

        def _acc(i, _):
            d = pl.ds(i * 16, 16)
            acc_v[d] += slot_v[d]
            return 0
        lax.fori_loop(0, SEGW * CPAD // 16, _acc, 0, unroll=8)
        return 0
    lax.fori_loop(0, 7, _slot, 0)

    iota16 = lax.iota(jnp.int32, 16)

    def _lbl(j, _):
        rowbase = (j * 16 + iota16) * CPAD
        best = plsc.load_gather(acc_v, [rowbase])
        besti = jnp.zeros((16,), jnp.int32)
        for cc in range(1, NCLS):
            v = plsc.load_gather(acc_v, [rowbase + cc])
            besti = jnp.where(v > best, cc, besti)
            best = jnp.maximum(v, best)
        lbl_v[pl.ds(j * 16, 16)] = besti.astype(jnp.float32)
        return 0
    lax.fori_loop(0, SEGW // 16, _lbl, 0)
    pltpu.sync_copy(lbl_v,
                    lbl_hbm.at[pl.ds(img_id * NTOK + (s % 8) * SEGW, SEGW)])


def _sc_call(seg_flat, gts_flat):
    mesh = plsc.VectorSubcoreMesh(core_axis_name="c", subcore_axis_name="s")
    fn = pl.kernel(
        _sc_body, mesh=mesh,
        compiler_params=pltpu.CompilerParams(needs_layout_passes=False),
        out_type=[
            jax.ShapeDtypeStruct((NPIX,), jnp.int32),
            jax.ShapeDtypeStruct((B * NTOK,), jnp.float32),
        ],
        scratch_types=[
            pltpu.VMEM((CHUNK,), jnp.int32),
            pltpu.VMEM((CHUNK,), jnp.int32),
            pltpu.VMEM((CHUNK,), jnp.int32),
            pltpu.VMEM((TBL,), jnp.float32),
            pltpu.VMEM((SEGW * CPAD,), jnp.float32),
            pltpu.VMEM((SEGW * CPAD,), jnp.float32),
            pltpu.VMEM((SEGW,), jnp.float32),
            pltpu.VMEM_SHARED((16, TBL), jnp.float32),
        ],
    )
    return fn(seg_flat, gts_flat)


def _main_body(img_ref, seg_ref, w1_ref, w2_ref,
               cnn_ref, tok_ref, acc_ref):
    t = pl.program_id(1)

    @pl.when(t == 0)
    def _init():
        acc_ref[...] = jnp.zeros_like(acc_ref)

    img_r = img_ref[0].reshape(CIN, P).astype(jnp.bfloat16)
    feats_t = jax.nn.relu(
        jax.lax.dot_general(w1_ref[...].astype(jnp.bfloat16), img_r,
                            (((0,), (0,)), ((), ())),
                            preferred_element_type=jnp.float32))
    cnn = jax.lax.dot_general(w2_ref[...], feats_t,
                              (((0,), (0,)), ((), ())),
                              preferred_element_type=jnp.float32)
    cnn_ref[0] = cnn.reshape(NCLS, ROWS, W)

    seg_row = seg_ref[0].reshape(1, P)
    iota_s = jax.lax.broadcasted_iota(jnp.int32, (NTOK, 1), 0)
    onehot = (iota_s == seg_row).astype(jnp.float8_e4m3fn)

    feats_b = feats_t.astype(jnp.float8_e4m3fn)
    ones_row = jnp.ones((1, P), jnp.float8_e4m3fn)
    rhs = jnp.concatenate([feats_b, ones_row], axis=0)
    acc_ref[...] += jax.lax.dot_general(
        onehot, rhs, (((1,), (1,)), ((), ())),
        preferred_element_type=jnp.float32)

    @pl.when(t == NT - 1)
    def _finalize():
        counts = acc_ref[:, CF:CF + 1]
        tok_ref[0] = acc_ref[:, :CF] / jnp.maximum(counts, 1.0)


def _attn_body(tok_ref, valid_ref, wq_ref, wk_ref, wv_ref, wo_ref, out_ref):
    tok = tok_ref[0]
    q = jnp.dot(tok, wq_ref[...], preferred_element_type=jnp.float32)
    k = jnp.dot(tok, wk_ref[...], preferred_element_type=jnp.float32)
    v = jnp.dot(tok, wv_ref[...], preferred_element_type=jnp.float32)
    s = jax.lax.dot_general(q, k, (((1,), (1,)), ((), ())),
                            preferred_element_type=jnp.float32)
    s = s * (1.0 / jnp.sqrt(jnp.float32(CF)))
    m = jnp.max(s, axis=1, keepdims=True)
    e = jnp.exp(s - m)
    p = e / jnp.sum(e, axis=1, keepdims=True)
    valid = valid_ref[0, 0]
    p = p * valid.reshape(MAXLEN, 1) * valid.reshape(1, MAXLEN)
    ctx = jnp.dot(p, v, preferred_element_type=jnp.float32)
    out_ref[0] = jnp.dot(ctx, wo_ref[...],
                         preferred_element_type=jnp.float32)


def kernel(img, gts, segments, n_tokens, W1, W2, Wq, Wk, Wv, Wo):
    sg_flat, lbl_flat = _sc_call(segments.reshape(-1), gts.reshape(-1))
    seg_global = sg_flat.reshape(B, H, W)
    super_labels = lbl_flat.reshape(B, NTOK)

    cnn_logits, tokens = pl.pallas_call(
        _main_body,
        grid=(B, NT),
        in_specs=[
            pl.BlockSpec((1, CIN, ROWS, W), lambda b, t: (b, 0, t, 0)),
            pl.BlockSpec((1, ROWS, W), lambda b, t: (b, t, 0)),
            pl.BlockSpec((CIN, CF), lambda b, t: (0, 0)),
            pl.BlockSpec((CF, NCLS), lambda b, t: (0, 0)),
        ],
        out_specs=[
            pl.BlockSpec((1, NCLS, ROWS, W), lambda b, t: (b, 0, t, 0)),
            pl.BlockSpec((1, NTOK, CF), lambda b, t: (b, 0, 0)),
        ],
        out_shape=[
            jax.ShapeDtypeStruct((B, NCLS, H, W), jnp.float32),
            jax.ShapeDtypeStruct((B, NTOK, CF), jnp.float32),
        ],
        scratch_shapes=[
            pltpu.VMEM((NTOK, CF + 1), jnp.float32),
        ],
    )(img, segments, W1, W2)

    valid = (jnp.arange(MAXLEN)[None, :] < n_tokens[:, None]).astype(jnp.float32)

    trans_logits = pl.pallas_call(
        _attn_body,
        grid=(B,),
        in_specs=[
            pl.BlockSpec((1, MAXLEN, CF), lambda b: (b, 0, 0)),
            pl.BlockSpec((1, 1, MAXLEN), lambda b: (b, 0, 0)),
            pl.BlockSpec((CF, CF), lambda b: (0, 0)),
            pl.BlockSpec((CF, CF), lambda b: (0, 0)),
            pl.BlockSpec((CF, CF), lambda b: (0, 0)),
            pl.BlockSpec((CF, NCLS), lambda b: (0, 0)),
        ],
        out_specs=pl.BlockSpec((1, MAXLEN, NCLS), lambda b: (b, 0, 0)),
        out_shape=jax.ShapeDtypeStruct((B, MAXLEN, NCLS), jnp.float32),
    )(tokens, valid.reshape(B, 1, MAXLEN), Wq, Wk, Wv, Wo)

    tokens_ids = jnp.arange(1, B * NTOK + 1)
    return (cnn_logits, trans_logits, super_labels, valid, tokens_ids,
            seg_global)


# speedup vs baseline: 1.0374x; 1.0367x over previous
import functools

import jax
import jax.numpy as jnp
from jax import lax
from jax.experimental import pallas as pl
from jax.experimental.pallas import tpu as pltpu
from jax.experimental.pallas import tpu_sc as plsc

B, H, W = 4, 384, 384
CIN, CF, NCLS = 3, 96, 10
MAXLEN = 512
NTOK = 512
ROWS = 32
P = ROWS * W
NT = H // ROWS

NPIX = B * H * W
WORKERS = 32
PPW = NPIX // WORKERS
CROWS = 24
NCHUNK = PPW // (CROWS * W)
CPAD = 16
TBL = NTOK * CPAD
SEGW = NTOK // 8


def _sc_body(seg_hbm, gts_hbm, sgout_hbm, lbl_hbm,
             seg_v, gts_v, sgo_v, table_v, acc_v, slot_v, lbl_v, shared):
    c = lax.axis_index("c")
    s = lax.axis_index("s")
    img_id = 2 * c + s // 8
    row_base = (s % 8) * (PPW // W)

    def _zero(i, _):
        table_v[pl.ds(i * 16, 16)] = jnp.zeros((16,), jnp.float32)
        return 0
    lax.fori_loop(0, TBL // 16, _zero, 0, unroll=8)

    ones16 = jnp.ones((16,), jnp.float32)
    off1 = img_id * NTOK + 1

    def _chunk(k, _):
        rb = row_base + k * CROWS
        pltpu.sync_copy(seg_hbm.at[img_id, pl.ds(rb, CROWS)], seg_v)
        pltpu.sync_copy(gts_hbm.at[img_id, pl.ds(rb, CROWS)], gts_v)

        def _row(r, _):
            def _col(j, _):
                d = pl.ds(j * 16, 16)
                sv = seg_v[r, d]
                gv = gts_v[r, d]
                sgo_v[r, d] = sv + off1
                plsc.addupdate_scatter(table_v, [sv * CPAD + gv], ones16)
                return 0
            lax.fori_loop(0, W // 16, _col, 0, unroll=8)
            return 0
        lax.fori_loop(0, CROWS, _row, 0)
        pltpu.sync_copy(sgo_v, sgout_hbm.at[img_id, pl.ds(rb, CROWS)])
        return 0
    lax.fori_loop(0, NCHUNK, _chunk, 0)

    pltpu.sync_copy(table_v, shared.at[s])
    plsc.subcore_barrier()

    srcbase = (s // 8) * 8
    segoff = (s % 8) * SEGW * CPAD
    pltpu.sync_copy(shared.at[srcbase, pl.ds(segoff, SEGW * CPAD)], acc_v)

    def _slot(t, _):
        pltpu.sync_copy(shared.at[srcbase + 1 + t, pl.ds(segoff, SEGW * CPAD)],
                        slot_v)

        def _acc(i, _):
            d = pl.ds(i * 16, 16)
            acc_v[d] += slot_v[d]
            return 0
        lax.fori_loop(0, SEGW * CPAD // 16, _acc, 0, unroll=8)
        return 0
    lax.fori_loop(0, 7, _slot, 0)

    iota16 = lax.iota(jnp.int32, 16)

    def _lbl(j, _):
        rowbase = (j * 16 + iota16) * CPAD
        best = plsc.load_gather(acc_v, [rowbase])
        besti = jnp.zeros((16,), jnp.int32)
        for cc in range(1, NCLS):
            v = plsc.load_gather(acc_v, [rowbase + cc])
            besti = jnp.where(v > best, cc, besti)
            best = jnp.maximum(v, best)
        lbl_v[pl.ds(j * 16, 16)] = besti.astype(jnp.float32)
        return 0
    lax.fori_loop(0, SEGW // 16, _lbl, 0)
    pltpu.sync_copy(lbl_v,
                    lbl_hbm.at[pl.ds(img_id * NTOK + (s % 8) * SEGW, SEGW)])


def _sc_call(seg_flat, gts_flat):
    mesh = plsc.VectorSubcoreMesh(core_axis_name="c", subcore_axis_name="s")
    fn = pl.kernel(
        _sc_body, mesh=mesh,
        compiler_params=pltpu.CompilerParams(needs_layout_passes=False),
        out_type=[
            jax.ShapeDtypeStruct((B, H, W), jnp.int32),
            jax.ShapeDtypeStruct((B * NTOK,), jnp.float32),
        ],
        scratch_types=[
            pltpu.VMEM((CROWS, W), jnp.int32),
            pltpu.VMEM((CROWS, W), jnp.int32),
            pltpu.VMEM((CROWS, W), jnp.int32),
            pltpu.VMEM((TBL,), jnp.float32),
            pltpu.VMEM((SEGW * CPAD,), jnp.float32),
            pltpu.VMEM((SEGW * CPAD,), jnp.float32),
            pltpu.VMEM((SEGW,), jnp.float32),
            pltpu.VMEM_SHARED((16, TBL), jnp.float32),
        ],
    )
    return fn(seg_flat, gts_flat)


def _main_body(img_ref, seg_ref, w1_ref, w2_ref,
               cnn_ref, tok_ref, acc_ref):
    t = pl.program_id(1)

    @pl.when(t == 0)
    def _init():
        acc_ref[...] = jnp.zeros_like(acc_ref)

    img_r = img_ref[0].reshape(CIN, P).astype(jnp.bfloat16)
    feats_t = jax.nn.relu(
        jax.lax.dot_general(w1_ref[...].astype(jnp.bfloat16), img_r,
                            (((0,), (0,)), ((), ())),
                            preferred_element_type=jnp.float32))
    cnn = jax.lax.dot_general(w2_ref[...], feats_t,
                              (((0,), (0,)), ((), ())),
                              preferred_element_type=jnp.float32)
    cnn_ref[0] = cnn.reshape(NCLS, ROWS, W)

    seg_row = seg_ref[0].reshape(1, P)
    iota_s = jax.lax.broadcasted_iota(jnp.int32, (NTOK, 1), 0)
    onehot = (iota_s == seg_row).astype(jnp.float8_e4m3fn)

    feats_b = feats_t.astype(jnp.float8_e4m3fn)
    ones_row = jnp.ones((1, P), jnp.float8_e4m3fn)
    rhs = jnp.concatenate([feats_b, ones_row], axis=0)
    acc_ref[...] += jax.lax.dot_general(
        onehot, rhs, (((1,), (1,)), ((), ())),
        preferred_element_type=jnp.float32)

    @pl.when(t == NT - 1)
    def _finalize():
        counts = acc_ref[:, CF:CF + 1]
        tok_ref[0] = acc_ref[:, :CF] / jnp.maximum(counts, 1.0)


def _attn_body(tok_ref, valid_ref, wq_ref, wk_ref, wv_ref, wo_ref, out_ref):
    tok = tok_ref[0]
    q = jnp.dot(tok, wq_ref[...], preferred_element_type=jnp.float32)
    k = jnp.dot(tok, wk_ref[...], preferred_element_type=jnp.float32)
    v = jnp.dot(tok, wv_ref[...], preferred_element_type=jnp.float32)
    s = jax.lax.dot_general(q, k, (((1,), (1,)), ((), ())),
                            preferred_element_type=jnp.float32)
    s = s * (1.0 / jnp.sqrt(jnp.float32(CF)))
    m = jnp.max(s, axis=1, keepdims=True)
    e = jnp.exp(s - m)
    p = e / jnp.sum(e, axis=1, keepdims=True)
    valid = valid_ref[0, 0]
    p = p * valid.reshape(MAXLEN, 1) * valid.reshape(1, MAXLEN)
    ctx = jnp.dot(p, v, preferred_element_type=jnp.float32)
    out_ref[0] = jnp.dot(ctx, wo_ref[...],
                         preferred_element_type=jnp.float32)


def kernel(img, gts, segments, n_tokens, W1, W2, Wq, Wk, Wv, Wo):
    seg_global, lbl_flat = _sc_call(segments, gts)
    super_labels = lbl_flat.reshape(B, NTOK)

    cnn_logits, tokens = pl.pallas_call(
        _main_body,
        grid=(B, NT),
        in_specs=[
            pl.BlockSpec((1, CIN, ROWS, W), lambda b, t: (b, 0, t, 0)),
            pl.BlockSpec((1, ROWS, W), lambda b, t: (b, t, 0)),
            pl.BlockSpec((CIN, CF), lambda b, t: (0, 0)),
            pl.BlockSpec((CF, NCLS), lambda b, t: (0, 0)),
        ],
        out_specs=[
            pl.BlockSpec((1, NCLS, ROWS, W), lambda b, t: (b, 0, t, 0)),
            pl.BlockSpec((1, NTOK, CF), lambda b, t: (b, 0, 0)),
        ],
        out_shape=[
            jax.ShapeDtypeStruct((B, NCLS, H, W), jnp.float32),
            jax.ShapeDtypeStruct((B, NTOK, CF), jnp.float32),
        ],
        scratch_shapes=[
            pltpu.VMEM((NTOK, CF + 1), jnp.float32),
        ],
    )(img, segments, W1, W2)

    valid = (jnp.arange(MAXLEN)[None, :] < n_tokens[:, None]).astype(jnp.float32)

    trans_logits = pl.pallas_call(
        _attn_body,
        grid=(B,),
        in_specs=[
            pl.BlockSpec((1, MAXLEN, CF), lambda b: (b, 0, 0)),
            pl.BlockSpec((1, 1, MAXLEN), lambda b: (b, 0, 0)),
            pl.BlockSpec((CF, CF), lambda b: (0, 0)),
            pl.BlockSpec((CF, CF), lambda b: (0, 0)),
            pl.BlockSpec((CF, CF), lambda b: (0, 0)),
            pl.BlockSpec((CF, NCLS), lambda b: (0, 0)),
        ],
        out_specs=pl.BlockSpec((1, MAXLEN, NCLS), lambda b: (b, 0, 0)),
        out_shape=jax.ShapeDtypeStruct((B, MAXLEN, NCLS), jnp.float32),
    )(tokens, valid.reshape(B, 1, MAXLEN), Wq, Wk, Wv, Wo)

    tokens_ids = jnp.arange(1, B * NTOK + 1)
    return (cnn_logits, trans_logits, super_labels, valid, tokens_ids,
            seg_global)


# speedup vs baseline: 1.0418x; 1.0043x over previous
import functools

import jax
import jax.numpy as jnp
from jax import lax
from jax.experimental import pallas as pl
from jax.experimental.pallas import tpu as pltpu
from jax.experimental.pallas import tpu_sc as plsc

B, H, W = 4, 384, 384
CIN, CF, NCLS = 3, 96, 10
MAXLEN = 512
NTOK = 512
ROWS = 32
P = ROWS * W
NT = H // ROWS

NPIX = B * H * W
WORKERS = 32
PPW = NPIX // WORKERS
CROWS = 24
NCHUNK = PPW // (CROWS * W)
CPAD = 16
TBL = NTOK * CPAD
SEGW = NTOK // 8


def _sc_body(seg_hbm, gts_hbm, sgout_hbm, lbl_hbm,
             seg_v, gts_v, sgo_v, table_v, acc_v, slot_v, lbl_v, shared):
    c = lax.axis_index("c")
    s = lax.axis_index("s")
    img_id = 2 * c + s // 8
    row_base = (s % 8) * (PPW // W)

    def _zero(i, _):
        table_v[pl.ds(i * 16, 16)] = jnp.zeros((16,), jnp.float32)
        return 0
    lax.fori_loop(0, TBL // 16, _zero, 0, unroll=8)

    ones16 = jnp.ones((16,), jnp.float32)
    off1 = img_id * NTOK + 1

    def _chunk(k, _):
        rb = row_base + k * CROWS
        pltpu.sync_copy(seg_hbm.at[img_id, pl.ds(rb, CROWS)], seg_v)
        pltpu.sync_copy(gts_hbm.at[img_id, pl.ds(rb, CROWS)], gts_v)

        def _row(r, _):
            def _col(j, _):
                d = pl.ds(j * 16, 16)
                sv = seg_v[r, d]
                gv = gts_v[r, d]
                sgo_v[r, d] = sv + off1
                plsc.addupdate_scatter(table_v, [sv * CPAD + gv], ones16)
                return 0
            lax.fori_loop(0, W // 16, _col, 0, unroll=8)
            return 0
        lax.fori_loop(0, CROWS, _row, 0)
        pltpu.sync_copy(sgo_v, sgout_hbm.at[img_id, pl.ds(rb, CROWS)])
        return 0
    lax.fori_loop(0, NCHUNK, _chunk, 0)

    pltpu.sync_copy(table_v, shared.at[s])
    plsc.subcore_barrier()

    srcbase = (s // 8) * 8
    segoff = (s % 8) * SEGW * CPAD
    pltpu.sync_copy(shared.at[srcbase, pl.ds(segoff, SEGW * CPAD)], acc_v)

    def _slot(t, _):
        pltpu.sync_copy(shared.at[srcbase + 1 + t, pl.ds(segoff, SEGW * CPAD)],
                        slot_v)

        def _acc(i, _):
            d = pl.ds(i * 16, 16)
            acc_v[d] += slot_v[d]
            return 0
        lax.fori_loop(0, SEGW * CPAD // 16, _acc, 0, unroll=8)
        return 0
    lax.fori_loop(0, 7, _slot, 0)

    iota16 = lax.iota(jnp.int32, 16)

    def _lbl(j, _):
        rowbase = (j * 16 + iota16) * CPAD
        best = plsc.load_gather(acc_v, [rowbase])
        besti = jnp.zeros((16,), jnp.int32)
        for cc in range(1, NCLS):
            v = plsc.load_gather(acc_v, [rowbase + cc])
            besti = jnp.where(v > best, cc, besti)
            best = jnp.maximum(v, best)
        lbl_v[pl.ds(j * 16, 16)] = besti.astype(jnp.float32)
        return 0
    lax.fori_loop(0, SEGW // 16, _lbl, 0)
    pltpu.sync_copy(lbl_v,
                    lbl_hbm.at[pl.ds(img_id * NTOK + (s % 8) * SEGW, SEGW)])


def _sc_call(seg_flat, gts_flat):
    mesh = plsc.VectorSubcoreMesh(core_axis_name="c", subcore_axis_name="s")
    fn = pl.kernel(
        _sc_body, mesh=mesh,
        compiler_params=pltpu.CompilerParams(needs_layout_passes=False),
        out_type=[
            jax.ShapeDtypeStruct((B, H, W), jnp.int32),
            jax.ShapeDtypeStruct((B * NTOK,), jnp.float32),
        ],
        scratch_types=[
            pltpu.VMEM((CROWS, W), jnp.int32),
            pltpu.VMEM((CROWS, W), jnp.int32),
            pltpu.VMEM((CROWS, W), jnp.int32),
            pltpu.VMEM((TBL,), jnp.float32),
            pltpu.VMEM((SEGW * CPAD,), jnp.float32),
            pltpu.VMEM((SEGW * CPAD,), jnp.float32),
            pltpu.VMEM((SEGW,), jnp.float32),
            pltpu.VMEM_SHARED((16, TBL), jnp.float32),
        ],
    )
    return fn(seg_flat, gts_flat)


def _main_body(img_ref, seg_ref, w1_ref, w2_ref,
               valid_ref, wq_ref, wk_ref, wv_ref, wo_ref,
               cnn_ref, trans_ref, acc_ref):
    t = pl.program_id(1)

    @pl.when(t == 0)
    def _init():
        acc_ref[...] = jnp.zeros_like(acc_ref)

    img_r = img_ref[0].reshape(CIN, P).astype(jnp.bfloat16)
    feats_t = jax.nn.relu(
        jax.lax.dot_general(w1_ref[...].astype(jnp.bfloat16), img_r,
                            (((0,), (0,)), ((), ())),
                            preferred_element_type=jnp.float32))
    cnn = jax.lax.dot_general(w2_ref[...], feats_t,
                              (((0,), (0,)), ((), ())),
                              preferred_element_type=jnp.float32)
    cnn_ref[0] = cnn.reshape(NCLS, ROWS, W)

    seg_row = seg_ref[0].reshape(1, P)
    iota_s = jax.lax.broadcasted_iota(jnp.int32, (NTOK, 1), 0)
    onehot = (iota_s == seg_row).astype(jnp.float8_e4m3fn)

    feats_b = feats_t.astype(jnp.float8_e4m3fn)
    ones_row = jnp.ones((1, P), jnp.float8_e4m3fn)
    rhs = jnp.concatenate([feats_b, ones_row], axis=0)
    acc_ref[...] += jax.lax.dot_general(
        onehot, rhs, (((1,), (1,)), ((), ())),
        preferred_element_type=jnp.float32)

    @pl.when(t == NT - 1)
    def _finalize():
        counts = acc_ref[:, CF:CF + 1]
        tok = acc_ref[:, :CF] / jnp.maximum(counts, 1.0)
        q = jnp.dot(tok, wq_ref[...], preferred_element_type=jnp.float32)
        k = jnp.dot(tok, wk_ref[...], preferred_element_type=jnp.float32)
        v = jnp.dot(tok, wv_ref[...], preferred_element_type=jnp.float32)
        sc = jax.lax.dot_general(q, k, (((1,), (1,)), ((), ())),
                                 preferred_element_type=jnp.float32)
        sc = sc * (1.0 / jnp.sqrt(jnp.float32(CF)))
        m = jnp.max(sc, axis=1, keepdims=True)
        e = jnp.exp(sc - m)
        p = e / jnp.sum(e, axis=1, keepdims=True)
        valid = valid_ref[0, 0]
        p = p * valid.reshape(MAXLEN, 1) * valid.reshape(1, MAXLEN)
        ctx = jnp.dot(p, v, preferred_element_type=jnp.float32)
        trans_ref[0] = jnp.dot(ctx, wo_ref[...],
                               preferred_element_type=jnp.float32)


def kernel(img, gts, segments, n_tokens, W1, W2, Wq, Wk, Wv, Wo):
    seg_global, lbl_flat = _sc_call(segments, gts)
    super_labels = lbl_flat.reshape(B, NTOK)

    valid = (jnp.arange(MAXLEN)[None, :] < n_tokens[:, None]).astype(jnp.float32)

    cnn_logits, trans_logits = pl.pallas_call(
        _main_body,
        grid=(B, NT),
        in_specs=[
            pl.BlockSpec((1, CIN, ROWS, W), lambda b, t: (b, 0, t, 0)),
            pl.BlockSpec((1, ROWS, W), lambda b, t: (b, t, 0)),
            pl.BlockSpec((CIN, CF), lambda b, t: (0, 0)),
            pl.BlockSpec((CF, NCLS), lambda b, t: (0, 0)),
            pl.BlockSpec((1, 1, MAXLEN), lambda b, t: (b, 0, 0)),
            pl.BlockSpec((CF, CF), lambda b, t: (0, 0)),
            pl.BlockSpec((CF, CF), lambda b, t: (0, 0)),
            pl.BlockSpec((CF, CF), lambda b, t: (0, 0)),
            pl.BlockSpec((CF, NCLS), lambda b, t: (0, 0)),
        ],
        out_specs=[
            pl.BlockSpec((1, NCLS, ROWS, W), lambda b, t: (b, 0, t, 0)),
            pl.BlockSpec((1, MAXLEN, NCLS), lambda b, t: (b, 0, 0)),
        ],
        out_shape=[
            jax.ShapeDtypeStruct((B, NCLS, H, W), jnp.float32),
            jax.ShapeDtypeStruct((B, MAXLEN, NCLS), jnp.float32),
        ],
        scratch_shapes=[
            pltpu.VMEM((NTOK, CF + 1), jnp.float32),
        ],
    )(img, segments, W1, W2, valid.reshape(B, 1, MAXLEN), Wq, Wk, Wv, Wo)

    tokens_ids = jnp.arange(1, B * NTOK + 1)
    return (cnn_logits, trans_logits, super_labels, valid, tokens_ids,
            seg_global)
